# Initial kernel scaffold; baseline (speedup 1.0000x reference)
import jax
import jax.numpy as jnp
from jax.experimental import pallas as pl


def kernel(x, params):
    raise NotImplementedError("write your pallas kernel here")


# speedup vs baseline: 7.7546x; 7.7546x over previous
import functools

import jax
import jax.numpy as jnp
from jax import lax
from jax.experimental import pallas as pl
from jax.experimental.pallas import tpu as pltpu
from jax.experimental.pallas import tpu_sc as plsc

_B = 8
_N = 1024
_BN = _B * _N
_K = 20
_KP = 32
_EPS = 1e-5
_NC = 2
_NS = 16
_NW = _NC * _NS
_PTS = _BN // _NW
_PBLK = 128


def _lrelu(z):
    return jnp.where(z >= 0, z, 0.2 * z)


def _bdot(a, b):
    return jnp.dot(a, b, preferred_element_type=jnp.float32)


def _prep_call(x, xsqc, xsqr):
    _, n, cp = x.shape

    def body(x_ref, xsqc_ref, xsqr_ref, gidx_ref, q_ref, acc_ref):
        b = pl.program_id(0)
        xm = x_ref[0]
        xb = xm.astype(jnp.bfloat16)
        g = lax.dot_general(xb, xb, (((1,), (1,)), ((), ())),
                            preferred_element_type=jnp.float32)
        inner = -2.0 * g
        q_ref[...] = ((-xsqc_ref[0]) - inner) - xsqr_ref[0]
        ii = lax.broadcasted_iota(jnp.int32, (n, n), 0)
        rowt = lax.broadcasted_iota(jnp.int32, (_KP, n), 0)
        neg = jnp.float32(-jnp.inf)

        def rnd(t, carry):
            qv = q_ref[...]
            mx = jnp.max(qv, axis=0, keepdims=True)
            am = jnp.min(jnp.where(qv >= mx, ii, n), axis=0,
                         keepdims=True)
            acc_ref[...] = jnp.where(rowt == t, am, acc_ref[...])
            q_ref[...] = jnp.where(ii == am, neg, qv)
            return carry

        lax.fori_loop(0, _K, rnd, 0)
        gidx_ref[0] = acc_ref[...] + b * n

    return pl.pallas_call(
        body,
        grid=(x.shape[0],),
        in_specs=[
            pl.BlockSpec((1, n, cp), lambda b: (b, 0, 0)),
            pl.BlockSpec((1, n, 1), lambda b: (b, 0, 0)),
            pl.BlockSpec((1, 1, n), lambda b: (b, 0, 0)),
        ],
        out_specs=pl.BlockSpec((1, _KP, n), lambda b: (b, 0, 0)),
        out_shape=jax.ShapeDtypeStruct((x.shape[0], _KP, n), jnp.int32),
        scratch_shapes=[
            pltpu.VMEM((n, n), jnp.float32),
            pltpu.VMEM((_KP, n), jnp.int32),
        ],
    )(x, xsqc, xsqr)


def _sc_gather(x_flat, gidx_flat):
    cp = x_flat.shape[1]
    cpp = 1024 // cp
    cpk = cpp * _K
    nstep = _PTS // cpp
    mesh = plsc.VectorSubcoreMesh(core_axis_name="c", subcore_axis_name="s")

    @functools.partial(
        pl.kernel,
        mesh=mesh,
        out_type=jax.ShapeDtypeStruct((_BN * _K, cp), jnp.float32),
        scratch_types=[
            pltpu.VMEM((_PTS * _K,), jnp.int32),
            pltpu.VMEM((cpk, cp), jnp.float32),
            pltpu.VMEM((cpk, cp), jnp.float32),
            pltpu.SemaphoreType.DMA,
            pltpu.SemaphoreType.DMA,
        ],
    )
    def body(x_hbm, gidx_hbm, xg_hbm, idx_v, buf0, buf1, sem0, sem1):
        wid = lax.axis_index("s") * _NC + lax.axis_index("c")
        base = wid * _PTS
        pltpu.sync_copy(gidx_hbm.at[pl.ds(pl.multiple_of(base * _K, 8),
                                          _PTS * _K)], idx_v)
        pltpu.async_copy(x_hbm.at[idx_v.at[pl.ds(0, cpk)]], buf0, sem0)

        def pair(ip, carry):
            i0 = ip * 2
            off0 = pl.multiple_of(i0 * cpk, 8)
            off1 = pl.multiple_of((i0 + 1) * cpk, 8)
            pltpu.async_copy(x_hbm.at[idx_v.at[pl.ds(off1, cpk)]],
                             buf1, sem1)
            pltpu.make_async_copy(x_hbm.at[idx_v.at[pl.ds(off0, cpk)]],
                                  buf0, sem0).wait()
            pltpu.sync_copy(buf0, xg_hbm.at[pl.ds(
                pl.multiple_of(base * _K + i0 * cpk, 8), cpk)])

            @pl.when(i0 + 2 < nstep)
            def _():
                off2 = pl.multiple_of((i0 + 2) * cpk, 8)
                pltpu.async_copy(x_hbm.at[idx_v.at[pl.ds(off2, cpk)]],
                                 buf0, sem0)

            pltpu.make_async_copy(x_hbm.at[idx_v.at[pl.ds(off1, cpk)]],
                                  buf1, sem1).wait()
            pltpu.sync_copy(buf1, xg_hbm.at[pl.ds(
                pl.multiple_of(base * _K + (i0 + 1) * cpk, 8), cpk)])
            return carry

        lax.fori_loop(0, nstep // 2, pair, 0)

    return body(x_flat, gidx_flat)


def _edge_call(xg, xf, wcat):
    cp = xf.shape[1]
    hp = wcat.shape[1]

    def body(xg_ref, x_ref, w_ref, mx_ref, s1_ref, s2_ref):
        xv = x_ref[...]
        xg3 = xg_ref[...].reshape(_PBLK, _K, cp)
        d = (xg3 - xv[:, None, :]).astype(jnp.bfloat16)
        db = d.reshape(_PBLK * _K, cp)
        xrep = jnp.broadcast_to(xv.astype(jnp.bfloat16)[:, None, :],
                                (_PBLK, _K, cp)).reshape(_PBLK * _K, cp)
        e = _bdot(jnp.concatenate([db, xrep], axis=1), w_ref[...])
        e3 = e.reshape(_PBLK, _K, hp)
        mx_ref[...] = jnp.max(e3, axis=1)
        s1_ref[...] = jnp.sum(e3, axis=1)
        s2_ref[...] = jnp.sum(e3 * e3, axis=1)

    nblk = _BN // _PBLK
    outspec = pl.BlockSpec((_PBLK, hp), lambda i: (i, 0))
    oshape = jax.ShapeDtypeStruct((_BN, hp), jnp.float32)
    return pl.pallas_call(
        body,
        grid=(nblk,),
        in_specs=[
            pl.BlockSpec((_PBLK * _K, cp), lambda i: (i, 0)),
            pl.BlockSpec((_PBLK, cp), lambda i: (i, 0)),
            pl.BlockSpec((2 * cp, hp), lambda i: (0, 0)),
        ],
        out_specs=[outspec, outspec, outspec],
        out_shape=[oshape, oshape, oshape],
    )(xg, xf, wcat)


def _finish_call(mx, s1, s2, gam, bet):
    hp = mx.shape[1]

    def body(mx_ref, s1_ref, s2_ref, g_ref, b_ref, o_ref):
        cnt = jnp.float32(_BN * _K)
        mean = jnp.sum(s1_ref[...], 0, keepdims=True) / cnt
        var = jnp.sum(s2_ref[...], 0, keepdims=True) / cnt - mean * mean
        sd = jnp.sqrt(var + _EPS)
        o_ref[...] = _lrelu(
            (mx_ref[...] - mean) / sd * g_ref[...] + b_ref[...])

    full = pl.BlockSpec((_BN, hp), lambda: (0, 0))
    vec = pl.BlockSpec((1, hp), lambda: (0, 0))
    return pl.pallas_call(
        body,
        in_specs=[full, full, full, vec, vec],
        out_specs=full,
        out_shape=jax.ShapeDtypeStruct((_BN, hp), jnp.float32),
    )(mx, s1, s2, gam, bet)


def _head_call(x1, x2, x3, x4, w5a, w5b, w5c, w5d, g5, b5,
               wf1, bf1, g6, b6, wf2, bf2, g7, b7, wf3, bf3):
    def body(x1_ref, x2_ref, x3_ref, x4_ref, w5a_ref, w5b_ref, w5c_ref,
             w5d_ref, g5_ref, b5_ref, wf1_ref, bf1_ref, g6_ref, b6_ref,
             wf2_ref, bf2_ref, g7_ref, b7_ref, wf3_ref, bf3_ref, o_ref):
        def bdot16(a_ref, w_ref):
            return _bdot(a_ref[...].astype(jnp.bfloat16), w_ref[...])

        y = (bdot16(x1_ref, w5a_ref) + bdot16(x2_ref, w5b_ref)
             + bdot16(x3_ref, w5c_ref) + bdot16(x4_ref, w5d_ref))
        cnt = jnp.float32(_BN)
        m = jnp.sum(y, 0, keepdims=True) / cnt
        var = jnp.sum(y * y, 0, keepdims=True) / cnt - m * m
        y = _lrelu((y - m) / jnp.sqrt(var + _EPS) * g5_ref[...] + b5_ref[...])
        xm = jnp.concatenate(
            [jnp.max(y[bb * _N:(bb + 1) * _N], 0, keepdims=True)
             for bb in range(_B)], 0)

        def bn_rows(t):
            mm = jnp.sum(t, 0, keepdims=True) / _B
            vv = jnp.sum(t * t, 0, keepdims=True) / _B - mm * mm
            return mm, vv

        t = _bdot(xm.astype(jnp.bfloat16), wf1_ref[...]) + bf1_ref[...]
        mm, vv = bn_rows(t)
        t = _lrelu((t - mm) / jnp.sqrt(vv + _EPS) * g6_ref[...] + b6_ref[...])
        t = _bdot(t.astype(jnp.bfloat16), wf2_ref[...]) + bf2_ref[...]
        mm, vv = bn_rows(t)
        t = _lrelu((t - mm) / jnp.sqrt(vv + _EPS) * g7_ref[...] + b7_ref[...])
        o_ref[...] = _bdot(t.astype(jnp.bfloat16), wf3_ref[...]) + bf3_ref[...]

    def fs(a):
        return pl.BlockSpec(a.shape, lambda: (0,) * a.ndim)

    args = (x1, x2, x3, x4, w5a, w5b, w5c, w5d, g5, b5,
            wf1, bf1, g6, b6, wf2, bf2, g7, b7, wf3, bf3)
    return pl.pallas_call(
        body,
        in_specs=[fs(a) for a in args],
        out_specs=pl.BlockSpec((_B, 40), lambda: (0, 0)),
        out_shape=jax.ShapeDtypeStruct((_B, 40), jnp.float32),
    )(*args)


def _pad_to(a, shape):
    return jnp.pad(a, tuple((0, t - s) for s, t in zip(a.shape, shape)))


def _edge_layer(xin_flat, w, gam, bet):
    cp = xin_flat.shape[1]
    c = w.shape[1] // 2
    hp = max(w.shape[0], 128)
    wcat = jnp.concatenate(
        [_pad_to(w[:, :c].T, (cp, hp)), _pad_to(w[:, c:].T, (cp, hp))],
        axis=0).astype(jnp.bfloat16)
    gam = jnp.pad(gam, (0, hp - gam.shape[0])).reshape(1, hp)
    bet = jnp.pad(bet, (0, hp - bet.shape[0])).reshape(1, hp)
    xt = jnp.transpose(xin_flat.reshape(_B, _N, cp), (0, 2, 1))[:, :c, :]
    xsq = jnp.sum(xt ** 2, axis=1)
    gidx = _prep_call(xin_flat.reshape(_B, _N, cp),
                      xsq.reshape(_B, _N, 1), xsq.reshape(_B, 1, _N))
    gidx_flat = jnp.transpose(gidx, (0, 2, 1))[:, :, :_K].reshape(_BN * _K)
    xg = _sc_gather(xin_flat, gidx_flat)
    mx, s1, s2 = _edge_call(xg, xin_flat, wcat)
    return _finish_call(mx, s1, s2, gam, bet)


def kernel(x, params):
    p = params
    x = x.astype(jnp.float32)
    h_flat = _pad_to(x.reshape(_BN, 3), (_BN, 128))
    feats = []
    for wn, gn, bn in (("W1", "g1", "b1"), ("W2", "g2", "b2"),
                       ("W3", "g3", "b3"), ("W4", "g4", "b4")):
        h_flat = _edge_layer(h_flat, p[wn], p[gn], p[bn])
        feats.append(h_flat)
    x1f, x2f, x3f, x4f = feats
    w5 = p["W5"]
    w5a = _pad_to(w5[:, :64].T, (x1f.shape[1], 512)).astype(jnp.bfloat16)
    w5b = _pad_to(w5[:, 64:128].T, (x2f.shape[1], 512)).astype(jnp.bfloat16)
    w5c = w5[:, 128:256].T.astype(jnp.bfloat16)
    w5d = w5[:, 256:].T.astype(jnp.bfloat16)
    y = _head_call(
        x1f, x2f, x3f, x4f, w5a, w5b, w5c, w5d,
        p["g5"].reshape(1, -1), p["b5"].reshape(1, -1),
        p["Wf1"].T.astype(jnp.bfloat16), p["bf1"].reshape(1, -1),
        p["g6"].reshape(1, -1), p["b6"].reshape(1, -1),
        p["Wf2"].T.astype(jnp.bfloat16), p["bf2"].reshape(1, -1),
        p["g7"].reshape(1, -1), p["b7"].reshape(1, -1),
        p["Wf3"].T.astype(jnp.bfloat16), p["bf3"].reshape(1, -1),
    )
    return y
